# Initial kernel scaffold; baseline (speedup 1.0000x reference)
import jax
import jax.numpy as jnp
from jax.experimental import pallas as pl


def kernel(x, edge_index, edge_attr, batch, W0, W1, W2, W3, b0, b1, b2, b3, lin_W, lin_b):
    raise NotImplementedError("write your pallas kernel here")


# speedup vs baseline: 9.9808x; 9.9808x over previous
import functools

import jax
import jax.numpy as jnp
from jax import lax
from jax.experimental import pallas as pl
from jax.experimental.pallas import tpu as pltpu
from jax.experimental.pallas import tpu_sc as plsc

N = 10000
E = 320000
D = 128
G = 64

NC = 2
NS = 16
NW = NC * NS

CH = 128
EPW = 10112
KCH = EPW // CH
EPAD = NW * EPW
NPAD = 10240
RPT = NPAD // NS

NB = 10
RB = 1000

_mesh = plsc.VectorSubcoreMesh(core_axis_name="c", subcore_axis_name="s")


@functools.partial(
    pl.kernel,
    mesh=_mesh,
    out_type=jax.ShapeDtypeStruct((NC, NPAD, 16), jnp.float32),
    scratch_types=[
        pltpu.VMEM((KCH, CH), jnp.int32),
        pltpu.VMEM((CH, 16), jnp.float32),
        pltpu.VMEM_SHARED((NPAD, 16), jnp.float32),
    ],
)
def _deg_kernel(dst3, zeros8, ones8, out, dst_v, ones_v, acc):
    c = lax.axis_index("c")
    s = lax.axis_index("s")
    wid = c * NS + s
    pltpu.sync_copy(dst3.at[wid], dst_v)
    pltpu.sync_copy(ones8, ones_v)
    pltpu.sync_copy(zeros8, acc.at[pl.ds(s * RPT, RPT)])
    plsc.subcore_barrier()

    @pl.loop(0, KCH)
    def _chunk(k):
        pltpu.sync_copy(ones_v, acc.at[dst_v.at[k]], add=True)

    plsc.subcore_barrier()
    pltpu.sync_copy(acc.at[pl.ds(s * RPT, RPT)], out.at[c, pl.ds(s * RPT, RPT)])


@functools.partial(
    pl.kernel,
    mesh=_mesh,
    out_type=jax.ShapeDtypeStruct((NC, NPAD, D), jnp.float32),
    scratch_types=[
        pltpu.VMEM((KCH, CH), jnp.int32),
        pltpu.VMEM((KCH, CH), jnp.int32),
        pltpu.VMEM((CH, D), jnp.float32),
        pltpu.VMEM_SHARED((NPAD, D), jnp.float32),
        pltpu.SemaphoreType.DMA,
    ],
)
def _scatter_kernel(h2, src3, dst3, zeros, out,
                    src_v, dst_v, rows0, acc, sem0):
    c = lax.axis_index("c")
    s = lax.axis_index("s")
    wid = c * NS + s
    pltpu.sync_copy(src3.at[wid], src_v)
    pltpu.sync_copy(dst3.at[wid], dst_v)
    pltpu.sync_copy(zeros, acc.at[pl.ds(s * RPT, RPT)])
    plsc.subcore_barrier()

    @pl.loop(0, KCH)
    def _chunk(k):
        pltpu.async_copy(h2.at[src_v.at[k]], rows0, sem0).wait()
        pltpu.sync_copy(rows0, acc.at[dst_v.at[k]], add=True)

    plsc.subcore_barrier()
    pltpu.sync_copy(acc.at[pl.ds(s * RPT, RPT)], out.at[c, pl.ds(s * RPT, RPT)])


def _k0_body(x_ref, w_ref, degp_ref, dis_ref, h2_ref):
    deg = degp_ref[0, :, 0:1] + degp_ref[1, :, 0:1] + 1.0
    dis = lax.rsqrt(deg)
    dis_ref[...] = dis
    h = jnp.dot(x_ref[...], w_ref[...], preferred_element_type=jnp.float32)
    h2_ref[...] = h * dis


def _mid_body(sp_ref, h2p_ref, dis_ref, b_ref, w_ref, h2_ref):
    dis = dis_ref[...]
    agg = sp_ref[0] + sp_ref[1] + h2p_ref[...]
    xn = jnp.maximum(agg * dis + b_ref[...], 0.0)
    h2_ref[...] = jnp.dot(xn, w_ref[...], preferred_element_type=jnp.float32) * dis


def _pool_body(sp_ref, h2p_ref, dis_ref, b_ref, linw_ref, linb_ref, batch_ref,
               out_ref, sums, cnts):
    i = pl.program_id(0)
    dis = dis_ref[...]
    agg = sp_ref[0] + sp_ref[1] + h2p_ref[...]
    xn = jnp.maximum(agg * dis + b_ref[...], 0.0)
    z = jnp.dot(xn, linw_ref[...], preferred_element_type=jnp.float32)
    gids = lax.broadcasted_iota(jnp.int32, (G, RB), 0)
    onehot = (gids == batch_ref[0]).astype(jnp.float32)
    psum = jnp.dot(onehot, z, preferred_element_type=jnp.float32)
    pcnt = jnp.sum(onehot, axis=1, keepdims=True)

    @pl.when(i == 0)
    def _():
        sums[...] = jnp.zeros_like(sums)
        cnts[...] = jnp.zeros_like(cnts)

    sums[...] += psum
    cnts[...] += pcnt

    @pl.when(i == NB - 1)
    def _():
        out_ref[...] = sums[...] / jnp.maximum(cnts[...], 1.0) + linb_ref[...]


def _row_spec(width):
    return pl.BlockSpec((RB, width), lambda i: (i, 0))


def _part_spec(width):
    return pl.BlockSpec((NC, RB, width), lambda i: (0, i, 0))


def _full_spec(r, c):
    return pl.BlockSpec((r, c), lambda i: (0, 0))


_k0_call = pl.pallas_call(
    _k0_body,
    grid=(NB,),
    in_specs=[_row_spec(D), _full_spec(D, D), _part_spec(16)],
    out_specs=[_row_spec(1), _row_spec(D)],
    out_shape=[
        jax.ShapeDtypeStruct((N, 1), jnp.float32),
        jax.ShapeDtypeStruct((N, D), jnp.float32),
    ],
)

_mid_call = pl.pallas_call(
    _mid_body,
    grid=(NB,),
    in_specs=[_part_spec(D), _row_spec(D), _row_spec(1), _full_spec(1, D),
              _full_spec(D, D)],
    out_specs=[_row_spec(D)],
    out_shape=[jax.ShapeDtypeStruct((N, D), jnp.float32)],
)

_pool_call = pl.pallas_call(
    _pool_body,
    grid=(NB,),
    in_specs=[_part_spec(D), _row_spec(D), _row_spec(1), _full_spec(1, D),
              _full_spec(D, 1), _full_spec(1, 1),
              pl.BlockSpec((1, 1, RB), lambda i: (i, 0, 0))],
    out_specs=[_full_spec(G, 1)],
    out_shape=[jax.ShapeDtypeStruct((G, 1), jnp.float32)],
    scratch_shapes=[
        pltpu.VMEM((G, 1), jnp.float32),
        pltpu.VMEM((G, 1), jnp.float32),
    ],
)


def kernel(x, edge_index, edge_attr, batch,
           W0, W1, W2, W3, b0, b1, b2, b3, lin_W, lin_b):
    src = edge_index[0]
    dst = edge_index[1]
    pad = EPAD - E
    srcp = jnp.concatenate([src, jnp.zeros((pad,), jnp.int32)])
    dstp = jnp.concatenate([dst, jnp.full((pad,), N, jnp.int32)])
    src3 = srcp.reshape(NW, KCH, CH)
    dst3 = dstp.reshape(NW, KCH, CH)

    zeros_d = jnp.zeros((RPT, D), jnp.float32)
    zeros_16 = jnp.zeros((RPT, 16), jnp.float32)
    ones_16 = jnp.ones((CH, 16), jnp.float32)

    degp = _deg_kernel(dst3, zeros_16, ones_16)
    dis, h2 = _k0_call(x, W0, degp)

    for b, W_next in ((b0, W1), (b1, W2), (b2, W3)):
        sp = _scatter_kernel(h2, src3, dst3, zeros_d)
        (h2,) = _mid_call(sp, h2, dis, b.reshape(1, D), W_next)

    sp = _scatter_kernel(h2, src3, dst3, zeros_d)
    (out,) = _pool_call(sp, h2, dis, b3.reshape(1, D), lin_W,
                        lin_b.reshape(1, 1), batch.reshape(NB, 1, RB))
    return out
